# Initial kernel scaffold; baseline (speedup 1.0000x reference)
import jax
import jax.numpy as jnp
from jax.experimental import pallas as pl


def kernel(preds, labels):
    raise NotImplementedError("write your pallas kernel here")


# speedup vs baseline: 8.2849x; 8.2849x over previous
import functools

import jax
import jax.numpy as jnp
from jax.experimental import pallas as pl
from jax.experimental.pallas import tpu as pltpu

SMOOTH = 0.1


def _lsce_block(preds_ref, labels_ref, out_ref, *, n_tokens, num_classes):
    i = pl.program_id(0)

    @pl.when(i == 0)
    def _init():
        out_ref[...] = jnp.zeros_like(out_ref)

    x = preds_ref[...]
    tb = x.shape[0]
    m = jnp.max(x, axis=1, keepdims=True)
    s = jnp.sum(jnp.exp(x - m), axis=1, keepdims=True)
    total = jnp.sum(x, axis=1, keepdims=True)
    lse = m + jnp.log(s)

    labels = labels_ref[0, 0, :]
    col = jax.lax.broadcasted_iota(jnp.int32, x.shape, 1)
    xl = jnp.sum(jnp.where(col == labels[:, None], x, 0.0), axis=1,
                 keepdims=True)

    a = SMOOTH / (num_classes - 1)
    lp_label = xl - lse
    sum_lp = total - num_classes * lse
    loss_t = -(a * (sum_lp - lp_label) + (1.0 - SMOOTH) * lp_label)
    out_ref[...] += jnp.sum(loss_t, keepdims=True).reshape(1, 1) / n_tokens


def kernel(preds, labels):
    b, t, c = preds.shape
    n_tokens = b * t
    preds2 = preds.reshape(n_tokens, c)
    tb = 128
    n_blocks = n_tokens // tb
    labels3 = labels.reshape(n_blocks, 1, tb).astype(jnp.int32)

    out = pl.pallas_call(
        functools.partial(_lsce_block, n_tokens=n_tokens, num_classes=c),
        grid=(n_blocks,),
        in_specs=[
            pl.BlockSpec((tb, c), lambda i: (i, 0)),
            pl.BlockSpec((1, 1, tb), lambda i: (i, 0, 0)),
        ],
        out_specs=pl.BlockSpec((1, 1), lambda i: (0, 0)),
        out_shape=jax.ShapeDtypeStruct((1, 1), jnp.float32),
    )(preds2, labels3)
    return out[0, 0]
